# baseline (speedup 1.0000x reference)
import functools

import jax
import jax.numpy as jnp
from jax import lax
from jax.experimental import pallas as pl
from jax.experimental.pallas import tpu as pltpu
from jax.experimental.pallas import tpu_sc as plsc

B = 16384
N_CAT = 26
N_NUM = 13
VOCAB = 100000
EMB = 16
OUT = 128
IN_E = N_CAT * EMB

NC = 2
NS = 16
NW = NC * NS
PLANES = N_CAT * EMB
P_T = PLANES // NW
OCHUNK = 4096


UNROLL = 4


def _sc_gather_t(tabT, catT, planes, base):
    p_t = planes // NW
    mesh = plsc.VectorSubcoreMesh(core_axis_name="c", subcore_axis_name="s")

    @functools.partial(
        pl.kernel,
        mesh=mesh,
        out_type=(
            jax.ShapeDtypeStruct((planes, B), jnp.float32),
            jax.ShapeDtypeStruct((planes, 16), jnp.float32),
        ),
        scratch_types=[
            pltpu.VMEM((VOCAB,), jnp.float32),
            pltpu.VMEM((B,), jnp.int32),
            pltpu.VMEM((OCHUNK,), jnp.float32),
            pltpu.VMEM((OCHUNK,), jnp.float32),
            pltpu.VMEM((16,), jnp.float32),
            pltpu.SemaphoreType.DMA,
            pltpu.SemaphoreType.DMA,
            pltpu.SemaphoreType.DMA,
        ],
        compiler_params=pltpu.CompilerParams(
            use_tc_tiling_on_sc=False, needs_layout_passes=False),
    )
    def k(tab_hbm, idx_hbm, out_hbm, st_hbm,
          p_v, ix_v, o_v0, o_v1, sv, psem, isem, osem):
        wid = lax.axis_index("c") * NS + lax.axis_index("s")
        lane = lax.iota(jnp.int32, 16)
        obufs = (o_v0, o_v1)
        pend = [None, None]
        for j in range(p_t):
            r = wid * p_t + j
            f = (base + r) // EMB
            dp = pltpu.async_copy(tab_hbm.at[r], p_v, psem)
            di = pltpu.async_copy(idx_hbm.at[f], ix_v, isem)
            dp.wait()
            di.wait()
            zero = jnp.zeros((16,), jnp.float32)
            sacc = zero
            qacc = zero
            for cc in range(B // OCHUNK):
                ob = obufs[cc % 2]
                if pend[cc % 2] is not None:
                    pend[cc % 2].wait()
                    pend[cc % 2] = None

                def gath(k2, carry, _cc=cc, _ob=ob):
                    sa, qa = carry
                    for u in range(UNROLL):
                        pos = k2 * (16 * UNROLL) + u * 16
                        v = plsc.load_gather(
                            p_v, [ix_v[pl.ds(_cc * OCHUNK + pos, 16)]])
                        _ob[pl.ds(pos, 16)] = v
                        sa = sa + v
                        qa = qa + v * v
                    return (sa, qa)

                sacc, qacc = lax.fori_loop(
                    0, OCHUNK // (16 * UNROLL), gath, (sacc, qacc))
                pend[cc % 2] = pltpu.async_copy(
                    ob, out_hbm.at[r, pl.ds(cc * OCHUNK, OCHUNK)], osem)
            ssum = jnp.sum(sacc)
            ssq = jnp.sum(qacc)
            sv[...] = jnp.where(lane == 0, ssum,
                                jnp.where(lane == 1, ssq, 0.0))
            pltpu.sync_copy(sv, st_hbm.at[r])
        for pd in pend:
            if pd is not None:
                pd.wait()

    return k(tabT, catT)


BLK = 2048
G = B // BLK


E1 = 224
E2 = IN_E - E1


def _tc_tail_body(num_ref, e1_ref, e2_ref, st1, st2,
                  gn, ge1, ge2, bn, be1, be2, w1, w2a, w2b, bb,
                  out_ref, sn, sqn, se1, sqe1, se2, sqe2):
    p = pl.program_id(0)
    i = pl.program_id(1)

    @pl.when(p == 0)
    def _stats():
        nblk = num_ref[...]
        s1 = jnp.sum(nblk, axis=0, keepdims=True)
        q1 = jnp.sum(nblk * nblk, axis=0, keepdims=True)

        @pl.when(i == 0)
        def _():
            sn[...] = s1
            sqn[...] = q1

        @pl.when(i > 0)
        def _():
            sn[...] += s1
            sqn[...] += q1

        @pl.when(i == G - 1)
        def _():
            inv_b = 1.0 / B
            mn = sn[...] * inv_b
            vn = sqn[...] * inv_b - mn * mn
            scale_n = gn[...] * lax.rsqrt(vn + 1e-5)
            sn[...] = scale_n
            sqn[...] = bn[...] - mn * scale_n
            for st, ge, be, se, sqe in ((st1, ge1, be1, se1, sqe1),
                                        (st2, ge2, be2, se2, sqe2)):
                me = st[:, 0:1] * inv_b
                ve = st[:, 1:2] * inv_b - me * me
                scale_e = ge[...] * lax.rsqrt(ve + 1e-5)
                se[...] = scale_e
                sqe[...] = be[...] - me * scale_e

    @pl.when(p == 1)
    def _matmul():
        dn_t = (((0,), (1,)), ((), ()))
        h_n = jnp.maximum(num_ref[...] * sn[...] + sqn[...], 0.0)
        h_e1 = jnp.maximum(e1_ref[...] * se1[...] + sqe1[...], 0.0)
        h_e2 = jnp.maximum(e2_ref[...] * se2[...] + sqe2[...], 0.0)
        out_ref[...] = (
            lax.dot_general(h_n, w1[...], (((1,), (1,)), ((), ())),
                            preferred_element_type=jnp.float32,
                            precision=lax.Precision.HIGHEST)
            + lax.dot_general(h_e1, w2a[...], dn_t,
                              preferred_element_type=jnp.float32,
                              precision=lax.Precision.HIGHEST)
            + lax.dot_general(h_e2, w2b[...], dn_t,
                              preferred_element_type=jnp.float32,
                              precision=lax.Precision.HIGHEST)
            + bb[...]
        )


def _tc_tail(num, embT1, embT2, st1, st2,
             gn, ge1, ge2, bn, be1, be2, w1, w2a, w2b, bb):
    full = lambda shape: pl.BlockSpec(shape, lambda p, i: (0, 0))
    rowblk = lambda shape: pl.BlockSpec(shape, lambda p, i: (i, 0))
    colblk = lambda shape: pl.BlockSpec(shape, lambda p, i: (0, i * p))
    return pl.pallas_call(
        _tc_tail_body,
        grid=(2, G),
        in_specs=[
            rowblk((BLK, N_NUM)),
            colblk((E1, BLK)),
            colblk((E2, BLK)),
            full((E1, 16)),
            full((E2, 16)),
            full((1, N_NUM)),
            full((E1, 1)),
            full((E2, 1)),
            full((1, N_NUM)),
            full((E1, 1)),
            full((E2, 1)),
            full((OUT, N_NUM)),
            full((OUT, E1)),
            full((OUT, E2)),
            full((1, OUT)),
        ],
        out_specs=rowblk((BLK, OUT)),
        out_shape=jax.ShapeDtypeStruct((B, OUT), jnp.float32),
        scratch_shapes=[
            pltpu.VMEM((1, N_NUM), jnp.float32),
            pltpu.VMEM((1, N_NUM), jnp.float32),
            pltpu.VMEM((E1, 1), jnp.float32),
            pltpu.VMEM((E1, 1), jnp.float32),
            pltpu.VMEM((E2, 1), jnp.float32),
            pltpu.VMEM((E2, 1), jnp.float32),
        ],
    )(num, embT1, embT2, st1, st2,
      gn, ge1, ge2, bn, be1, be2, w1, w2a, w2b, bb)


def kernel(num, cat_indices, tables, gamma, beta, W, b):
    tabT = tables.transpose(0, 2, 1).reshape(PLANES, VOCAB)
    catT = cat_indices.T
    embT1, st1 = _sc_gather_t(tabT[:E1], catT, E1, 0)
    embT2, st2 = _sc_gather_t(tabT[E1:], catT, E2, E1)
    ge = gamma[N_NUM:]
    be = beta[N_NUM:]
    out = _tc_tail(
        num, embT1, embT2, st1, st2,
        gamma[:N_NUM].reshape(1, N_NUM),
        ge[:E1].reshape(E1, 1), ge[E1:].reshape(E2, 1),
        beta[:N_NUM].reshape(1, N_NUM),
        be[:E1].reshape(E1, 1), be[E1:].reshape(E2, 1),
        W[:, :N_NUM],
        W[:, N_NUM:N_NUM + E1], W[:, N_NUM + E1:],
        b.reshape(1, OUT),
    )
    return out
